# baseline (speedup 1.0000x reference)
import functools

import jax
import jax.numpy as jnp
from jax import lax
from jax.experimental import pallas as pl
from jax.experimental.pallas import tpu as pltpu
from jax.experimental.pallas import tpu_sc as plsc

N = 10000
E = 160000
D = 256


def _mlp_body(x_ref, w1_ref, b1_ref, w2_ref, b2_ref, w3_ref, b3_ref, o_ref):
    x = x_ref[...]
    h = jnp.dot(x, w1_ref[...], preferred_element_type=jnp.float32) + b1_ref[...]
    h = h * 0.5 * (1.0 + lax.erf(h * 0.7071067811865476))
    h = jnp.dot(h, w2_ref[...], preferred_element_type=jnp.float32) + b2_ref[...]
    h = jnp.dot(h, w3_ref[...], preferred_element_type=jnp.float32) + b3_ref[...]
    o_ref[...] = h


def _mlp_tc(x, W1, b1, W2, b2, W3, b3, block_rows):
    rows = x.shape[0]
    grid = rows // block_rows
    wspec = pl.BlockSpec((D, D), lambda i: (0, 0))
    bspec = pl.BlockSpec((1, D), lambda i: (0, 0))
    return pl.pallas_call(
        _mlp_body,
        grid=(grid,),
        in_specs=[
            pl.BlockSpec((block_rows, D), lambda i: (i, 0)),
            wspec, bspec, wspec, bspec, wspec, bspec,
        ],
        out_specs=pl.BlockSpec((block_rows, D), lambda i: (i, 0)),
        out_shape=jax.ShapeDtypeStruct((rows, D), jnp.float32),
    )(x, W1, b1.reshape(1, D), W2, b2.reshape(1, D), W3, b3.reshape(1, D))


def _bn_stats2_body(a_ref, b_ref, s_ref, q_ref):
    x = a_ref[...] if b_ref is None else a_ref[...] + b_ref[...]
    s = jnp.sum(x, axis=0, keepdims=True)
    q = jnp.sum(x * x, axis=0, keepdims=True)

    @pl.when(pl.program_id(0) == 0)
    def _():
        s_ref[...] = jnp.zeros_like(s_ref)
        q_ref[...] = jnp.zeros_like(q_ref)

    s_ref[...] += s
    q_ref[...] += q


def _bn_apply2_body(a_ref, b_ref, s_ref, q_ref, g_ref, be_ref, o_ref, *, count):
    mean = s_ref[...] / count
    var = q_ref[...] / count - mean * mean
    inv = lax.rsqrt(var + 1e-5)
    x = a_ref[...] if b_ref is None else a_ref[...] + b_ref[...]
    o_ref[...] = (x - mean) * (inv * g_ref[...]) + be_ref[...]


def _bn2_tc(a, b, gamma, beta, block_rows, out_rows=None):
    rows = a.shape[0] if out_rows is None else out_rows
    grid = rows // block_rows
    xspec = pl.BlockSpec((block_rows, D), lambda i: (i, 0))
    cspec = pl.BlockSpec((1, D), lambda i: (0, 0))
    ins = [a] if b is None else [a, b]
    in_x_specs = [xspec] * len(ins)

    def stats_body(*refs):
        if b is None:
            _bn_stats2_body(refs[0], None, refs[1], refs[2])
        else:
            _bn_stats2_body(*refs)

    def apply_body(*refs, count):
        if b is None:
            _bn_apply2_body(refs[0], None, *refs[1:], count=count)
        else:
            _bn_apply2_body(*refs, count=count)

    s, q = pl.pallas_call(
        stats_body,
        grid=(grid,),
        in_specs=in_x_specs,
        out_specs=[cspec, cspec],
        out_shape=[jax.ShapeDtypeStruct((1, D), jnp.float32)] * 2,
    )(*ins)
    return pl.pallas_call(
        functools.partial(apply_body, count=float(rows)),
        grid=(grid,),
        in_specs=in_x_specs + [cspec, cspec, cspec, cspec],
        out_specs=xspec,
        out_shape=jax.ShapeDtypeStruct((rows, D), jnp.float32),
    )(*ins, s, q, gamma.reshape(1, D), beta.reshape(1, D))


def _seg_max_fix_body(p_ref, o_ref):
    m = jnp.max(p_ref[...], axis=0, keepdims=True)
    o_ref[...] = jnp.where(m == -jnp.inf, 0.0, m)


def _seg_sum_body(p_ref, o_ref):
    o_ref[...] = jnp.sum(p_ref[...], axis=0, keepdims=True)


def _reduce_partials_tc(p, kind):
    body = _seg_max_fix_body if kind == "max" else _seg_sum_body
    w, n = p.shape
    return pl.pallas_call(
        body,
        grid=(1,),
        in_specs=[pl.BlockSpec((w, n), lambda i: (0, 0))],
        out_specs=pl.BlockSpec((1, n), lambda i: (0, 0)),
        out_shape=jax.ShapeDtypeStruct((1, n), jnp.float32),
    )(p)


NC = 2
NS = 16
NW = NC * NS
L = 16
PW = 5120
E_PAD = NW * PW
N_PAD = 10240
B = 64
NH = N // NC

_f32 = jnp.float32
_i32 = jnp.int32


def _sc_mesh():
    return plsc.VectorSubcoreMesh(core_axis_name="c", subcore_axis_name="s",
                                  num_cores=NC, num_subcores=NS)


_SC_PARAMS = pltpu.CompilerParams(use_tc_tiling_on_sc=False,
                                  needs_layout_passes=False)


def _lanes():
    return lax.broadcasted_iota(_i32, (L,), 0)


def _fill_1d(ref, n, value):
    def body(i, _):
        ref[pl.ds(i * L, L)] = jnp.full((L,), value, _f32)
        return 0
    lax.fori_loop(0, n // L, body, 0)


def _attn_max_sc(n_h, e_h, src, dst):
    @functools.partial(
        pl.kernel, mesh=_sc_mesh(), compiler_params=_SC_PARAMS,
        out_type=[jax.ShapeDtypeStruct((E_PAD,), _f32),
                  jax.ShapeDtypeStruct((NW, N_PAD), _f32)],
        scratch_types=[
            pltpu.VMEM((B,), _i32), pltpu.VMEM((B,), _i32),
            pltpu.VMEM((B, D), _f32), pltpu.VMEM((B, D), _f32),
            pltpu.VMEM((B, D), _f32), pltpu.VMEM((B,), _f32),
            pltpu.VMEM((N_PAD,), _f32),
            pltpu.SemaphoreType.DMA, pltpu.SemaphoreType.DMA,
        ])
    def k(nh_hbm, eh_hbm, src_hbm, dst_hbm, attn_hbm, maxp_hbm,
          sidx, didx, rows_s, rows_d, rows_e, attn_buf, maxarr, sem1, sem2):
        wid = lax.axis_index("c") * NS + lax.axis_index("s")
        _fill_1d(maxarr, N_PAD, -jnp.inf)

        def chunk(ci, _):
            base = wid * PW + ci * B
            rb = jnp.minimum(base, E - B)
            pltpu.sync_copy(src_hbm.at[pl.ds(rb, B)], sidx)
            pltpu.sync_copy(dst_hbm.at[pl.ds(rb, B)], didx)
            c1 = pltpu.async_copy(nh_hbm.at[sidx], rows_s, sem1)
            c2 = pltpu.async_copy(nh_hbm.at[didx], rows_d, sem2)
            pltpu.sync_copy(eh_hbm.at[pl.ds(rb, B)], rows_e)
            c1.wait()
            c2.wait()
            for v in range(B // L):
                eidx = jnp.full((L,), v * L, _i32) + _lanes()

                def dstep(t, acc):
                    for u in range(8):
                        col = jnp.full((L,), t * 8 + u, _i32)
                        sv = plsc.load_gather(rows_s, [eidx, col])
                        ev = plsc.load_gather(rows_e, [eidx, col])
                        dv = plsc.load_gather(rows_d, [eidx, col])
                        acc = acc + (sv + ev) * dv
                    return acc

                vec = lax.fori_loop(0, D // 8, dstep, jnp.zeros((L,), _f32))
                attn_buf[pl.ds(v * L, L)] = vec
                dstv = didx[pl.ds(v * L, L)]
                emask = (base + v * L + _lanes()) < E
                for _ in range(L):
                    cur = plsc.load_gather(maxarr, [dstv])
                    m = emask & (cur < vec)
                    plsc.store_scatter(maxarr, [dstv], vec, mask=m)
            pltpu.sync_copy(attn_buf, attn_hbm.at[pl.ds(base, B)])
            return 0

        lax.fori_loop(0, PW // B, chunk, 0)
        pltpu.sync_copy(maxarr, maxp_hbm.at[wid])

    return k(n_h, e_h, src, dst)


def _exp_den_sc(attn, seg_max, dst):
    @functools.partial(
        pl.kernel, mesh=_sc_mesh(), compiler_params=_SC_PARAMS,
        out_type=[jax.ShapeDtypeStruct((E_PAD,), _f32),
                  jax.ShapeDtypeStruct((NW, N_PAD), _f32)],
        scratch_types=[
            pltpu.VMEM((B,), _i32), pltpu.VMEM((B,), _f32),
            pltpu.VMEM((B,), _f32),
            pltpu.VMEM((N_PAD,), _f32), pltpu.VMEM((N_PAD,), _f32),
        ])
    def k(attn_hbm, mx_hbm, dst_hbm, ex_hbm, denp_hbm,
          didx, attn_b, ex_b, mxloc, denarr):
        wid = lax.axis_index("c") * NS + lax.axis_index("s")
        pltpu.sync_copy(mx_hbm, mxloc)
        _fill_1d(denarr, N_PAD, 0.0)
        lanes = _lanes()

        def chunk(ci, _):
            base = wid * PW + ci * B
            rb = jnp.minimum(base, E - B)
            pltpu.sync_copy(dst_hbm.at[pl.ds(rb, B)], didx)
            pltpu.sync_copy(attn_hbm.at[pl.ds(rb, B)], attn_b)
            for v in range(B // L):
                sl = pl.ds(v * L, L)
                dstv = didx[sl]
                mx = plsc.load_gather(mxloc, [dstv])
                exv = jnp.exp(attn_b[sl] - mx)
                ex_b[sl] = exv
                emask = (base + v * L + lanes) < E
                exm = jnp.where(emask, exv, 0.0)
                for j in range(L):
                    plsc.addupdate_scatter(denarr, [dstv], exm, mask=lanes == j)
            pltpu.sync_copy(ex_b, ex_hbm.at[pl.ds(base, B)])
            return 0

        lax.fori_loop(0, PW // B, chunk, 0)
        pltpu.sync_copy(denarr, denp_hbm.at[wid])

    return k(attn, seg_max, dst)


def _nz_sc(n_h, src_pad, dst_pad, ex, denom):
    EPT = E_PAD // NS
    ZR = 20
    DR = NH // NS + 1

    @functools.partial(
        pl.kernel, mesh=_sc_mesh(), compiler_params=_SC_PARAMS,
        out_type=jax.ShapeDtypeStruct((N, D), _f32),
        scratch_types=[
            pltpu.VMEM((512,), _i32), pltpu.VMEM((E_PAD // NS,), _i32),
            pltpu.VMEM((E_PAD // NS,), _f32),
            pltpu.VMEM((L, D), _f32), pltpu.VMEM((L, D), _f32),
            pltpu.VMEM((4, D), _f32),
            pltpu.VMEM((N_PAD,), _f32),
            pltpu.VMEM_SHARED((NS * NS * 20, D), _f32),
            pltpu.SemaphoreType.DMA,
        ])
    def k(nh_hbm, src_hbm, dst_hbm, ex_hbm, den_hbm, nz_hbm,
          sidx, didx, exb, rows_a, rows_b, zbuf, denloc, nzacc, sem):
        c = lax.axis_index("c")
        s = lax.axis_index("s")
        lo = c * NH
        lanes = _lanes()
        pltpu.sync_copy(den_hbm, denloc)
        pltpu.sync_copy(dst_hbm.at[pl.ds(s * EPT, EPT)], didx)
        pltpu.sync_copy(ex_hbm.at[pl.ds(s * EPT, EPT)], exb)
        for i in range(4):
            for dc in range(D // L):
                zbuf[i, pl.ds(dc * L, L)] = jnp.zeros((L,), _f32)
        for r in range(NS * ZR // 4):
            pltpu.sync_copy(zbuf, nzacc.at[pl.ds(s * (NS * ZR) + r * 4, 4)])
        plsc.subcore_barrier()
        SB = 512

        def chunk(ci, _):
            pltpu.sync_copy(src_hbm.at[pl.ds(s * EPT + ci * SB, SB)], sidx)

            def grp(v, _):
                off = ci * SB + v * L
                srcv = sidx[pl.ds(v * L, L)]
                cp = pltpu.async_copy(nh_hbm.at[srcv], rows_a, sem)
                dstv = didx[pl.ds(off, L)]
                emask = (s * EPT + off + lanes) < E
                own = (dstv >= lo) & (dstv < lo + NH) & emask
                ldst = jnp.where(own, dstv - lo, 0)
                cp.wait()
                for i in range(L):
                    col = jnp.full((L,), 1, _i32) * (off + i)
                    ds_s = plsc.load_gather(didx, [col])
                    own_s = (ds_s >= lo) & (ds_s < lo + NH)
                    eid_s = jnp.full((L,), 1, _i32) * (s * EPT + off + i)
                    m = own_s & (eid_s < E)
                    ex_s = plsc.load_gather(exb, [col])
                    den_i = jnp.where(m, ds_s, 0)
                    ws = ex_s / plsc.load_gather(denloc, [den_i])
                    ws = jnp.where(m, ws, 0.0)
                    for dc in range(D // L):
                        dsl = pl.ds(dc * L, L)
                        rows_b[i, dsl] = rows_a[i, dsl] * ws
                pltpu.sync_copy(rows_b, nzacc.at[ldst], add=True)
                return 0

            lax.fori_loop(0, SB // L, grp, 0)
            return 0

        lax.fori_loop(0, EPT // SB, chunk, 0)
        plsc.subcore_barrier()

        @pl.when(s < NS - 1)
        def _():
            pltpu.sync_copy(nzacc.at[pl.ds(s * DR, DR)],
                            nz_hbm.at[pl.ds(c * NH + s * DR, DR)])

        @pl.when(s == NS - 1)
        def _():
            rem = NH - (NS - 1) * DR
            pltpu.sync_copy(nzacc.at[pl.ds((NS - 1) * DR, rem)],
                            nz_hbm.at[pl.ds(c * NH + (NS - 1) * DR, rem)])

    return k(n_h, src_pad, dst_pad, ex, denom)


def _escale_sc(e_h, nz, src, dst):
    @functools.partial(
        pl.kernel, mesh=_sc_mesh(), compiler_params=_SC_PARAMS,
        out_type=jax.ShapeDtypeStruct((E_PAD, D), _f32),
        scratch_types=[
            pltpu.VMEM((B,), _i32), pltpu.VMEM((B,), _i32),
            pltpu.VMEM((B, D), _f32), pltpu.VMEM((B, D), _f32),
            pltpu.VMEM((L, D), _f32), pltpu.VMEM((L, D), _f32),
            pltpu.SemaphoreType.DMA, pltpu.SemaphoreType.DMA,
        ])
    def k(eh_hbm, nz_hbm, src_hbm, dst_hbm, esc_hbm,
          sidx, didx, ehb, outb, nzs, nzd, sem1, sem2):
        wid = lax.axis_index("c") * NS + lax.axis_index("s")

        def chunk(ci, _):
            base = wid * PW + ci * B
            rb = jnp.minimum(base, E - B)
            pltpu.sync_copy(src_hbm.at[pl.ds(rb, B)], sidx)
            pltpu.sync_copy(dst_hbm.at[pl.ds(rb, B)], didx)
            pltpu.sync_copy(eh_hbm.at[pl.ds(rb, B)], ehb)
            for v in range(B // L):
                sl = pl.ds(v * L, L)
                c1 = pltpu.async_copy(nz_hbm.at[sidx[sl]], nzs, sem1)
                c2 = pltpu.async_copy(nz_hbm.at[didx[sl]], nzd, sem2)
                c1.wait()
                c2.wait()
                for i in range(L):
                    e = v * L + i
                    for dc in range(D // L):
                        dsl = pl.ds(dc * L, L)
                        outb[e, dsl] = ehb[e, dsl] * (
                            1.0 + nzs[i, dsl] - nzd[i, dsl])
            pltpu.sync_copy(outb, esc_hbm.at[pl.ds(base, B)])
            return 0

        lax.fori_loop(0, PW // B, chunk, 0)

    return k(e_h, nz, src, dst)


def kernel(nh, eh, edge_index, nfW1, nfb1, nfW2, nfb2, nfW3, nfb3,
           efW1, efb1, efW2, efb2, efW3, efb3,
           nf_gamma, nf_beta, ef_gamma, ef_beta):
    src = edge_index[0].astype(jnp.int32)
    dst = edge_index[1].astype(jnp.int32)

    n_h = _mlp_tc(nh, nfW1, nfb1, nfW2, nfb2, nfW3, nfb3, 1000)
    e_h = _mlp_tc(eh, efW1, efb1, efW2, efb2, efW3, efb3, 2000)

    attn, maxp = _attn_max_sc(n_h, e_h, src, dst)
    seg_max = _reduce_partials_tc(maxp, "max").reshape(N_PAD)
    ex, denp = _exp_den_sc(attn, seg_max, dst)
    denom = _reduce_partials_tc(denp, "sum").reshape(N_PAD)
    src_pad = jnp.pad(src, (0, E_PAD - E))
    dst_pad = jnp.pad(dst, (0, E_PAD - E))
    nz = _nz_sc(n_h, src_pad, dst_pad, ex, denom)
    e_scaled = _escale_sc(e_h, nz, src, dst)

    n_out = _bn2_tc(n_h, nz, nf_gamma, nf_beta, 1000)
    e_out = _bn2_tc(e_scaled, None, ef_gamma, ef_beta, 2000, out_rows=E)
    return (n_out, e_out)
